# Initial kernel scaffold; baseline (speedup 1.0000x reference)
import functools

import numpy as np
import jax
import jax.numpy as jnp
from jax import lax
from jax.experimental import pallas as pl
from jax.experimental.pallas import tpu as pltpu
from jax.experimental.pallas import tpu_sc as plsc

_FIELD_DIM = 38462
_F = 26
_D = 16
_B = 16384
_NC = 2
_NS = 16
_NW = _NC * _NS
_BPW = _B // _NW
_CH = 64
_NCHUNK = _BPW // _CH
_IPC = _CH * _F
_GW = 128
_NG = _IPC // _GW


def _fm_body(x_hbm, pat_hbm, table_hbm, out_hbm,
             idx0, idx1, rows0, rows1, patv, out_v, sem0, sem1):
    wid = lax.axis_index("s") * _NC + lax.axis_index("c")
    xrow0 = wid * (_NG * _NCHUNK)

    pltpu.sync_copy(pat_hbm, patv)

    def load_and_fire(g, idxb, rows, sem):
        pltpu.sync_copy(x_hbm.at[pl.ds(xrow0 + g * _NG, _NG), :], idxb)

        def add_off(j, carry):
            for v in range(_GW // 16):
                sl = pl.ds(v * 16, 16)
                idxb[j, sl] = idxb[j, sl] + patv[j, sl]
            return carry
        lax.fori_loop(0, _NG, add_off, 0)

        handles = []
        for j in range(_NG):
            handles.append(
                pltpu.async_copy(table_hbm.at[idxb.at[j]],
                                 rows.at[pl.ds(j * _GW, _GW), :], sem))
        return handles

    def compute(g, rows):
        b0 = g * _CH

        def body(r, carry):
            base = r * _F
            e = rows[base, :]
            s = e
            q = e * e
            for f in range(1, _F):
                e = rows[base + f, :]
                s = s + e
                q = q + e * e
            a = s * s - q
            out_v[b0 + r] = 0.5 * jnp.sum(a)
            return carry
        lax.fori_loop(0, _CH, body, 0)

    bufs = ((idx0, rows0, sem0), (idx1, rows1, sem1))
    handles = load_and_fire(0, *bufs[0])
    prev_rows = rows0
    for g in range(1, _NCHUNK):
        idxb, rows, sem = bufs[g % 2]
        new_handles = load_and_fire(g, idxb, rows, sem)
        for h in handles:
            h.wait()
        compute(g - 1, prev_rows)
        handles, prev_rows = new_handles, rows
    for h in handles:
        h.wait()
    compute(_NCHUNK - 1, prev_rows)

    pltpu.sync_copy(out_v, out_hbm.at[pl.ds(wid * _BPW, _BPW)])


def kernel(x, table):
    pat = (np.arange(_IPC, dtype=np.int64) % _F) * _FIELD_DIM
    pat = jnp.asarray(pat.astype(np.int32).reshape(_NG, _GW))
    x2 = x.reshape(_B * _F // _GW, _GW)

    mesh = plsc.VectorSubcoreMesh(core_axis_name="c", subcore_axis_name="s")
    fm = functools.partial(
        pl.kernel,
        mesh=mesh,
        out_type=jax.ShapeDtypeStruct((_B,), jnp.float32),
        scratch_types=[
            pltpu.VMEM((_NG, _GW), jnp.int32),
            pltpu.VMEM((_NG, _GW), jnp.int32),
            pltpu.VMEM((_IPC, _D), jnp.float32),
            pltpu.VMEM((_IPC, _D), jnp.float32),
            pltpu.VMEM((_NG, _GW), jnp.int32),
            pltpu.VMEM((_BPW,), jnp.float32),
            pltpu.SemaphoreType.DMA,
            pltpu.SemaphoreType.DMA,
        ],
    )(_fm_body)
    return fm(x2, pat, table)


# speedup vs baseline: 1.2501x; 1.2501x over previous
import functools

import numpy as np
import jax
import jax.numpy as jnp
from jax import lax
from jax.experimental import pallas as pl
from jax.experimental.pallas import tpu as pltpu
from jax.experimental.pallas import tpu_sc as plsc

_FIELD_DIM = 38462
_F = 26
_D = 16
_B = 16384
_NC = 2
_NS = 16
_NW = _NC * _NS
_BPW = _B // _NW
_CH = 64
_NCHUNK = _BPW // _CH
_IPC = _CH * _F
_GW = 128
_NG = _IPC // _GW


def _fm_body(x_hbm, pat_hbm, table_hbm, out_hbm,
             xbuf, idx0, idx1, rows0, rows1, patv, out_v, sem0, sem1):
    wid = lax.axis_index("s") * _NC + lax.axis_index("c")
    x0 = wid * (_IPC * _NCHUNK)

    pltpu.sync_copy(pat_hbm, patv)

    def load_and_fire(g, idxb, rows, sem):
        pltpu.sync_copy(x_hbm.at[pl.ds(x0 + g * _IPC, _IPC)], xbuf)

        def add_off(j, carry):
            for v in range(_GW // 16):
                sl = pl.ds(j * _GW + v * 16, 16)
                idxb[j, pl.ds(v * 16, 16)] = xbuf[sl] + patv[sl]
            return carry
        lax.fori_loop(0, _NG, add_off, 0)

        handles = []
        for j in range(_NG):
            handles.append(
                pltpu.async_copy(table_hbm.at[idxb.at[j]],
                                 rows.at[pl.ds(j * _GW, _GW), :], sem))
        return handles

    def compute(g, rows):
        b0 = g * _CH
        lanes = lax.iota(jnp.int32, 16)

        def group(t, carry):
            gbase = t * 16

            def body(r, acc):
                base = (gbase + r) * _F
                e = rows[base, :]
                s = e
                q = e * e
                for f in range(1, _F):
                    e = rows[base + f, :]
                    s = s + e
                    q = q + e * e
                a = s * s - q
                for sh in (8, 4, 2, 1):
                    a = a + a.at[lanes ^ sh].get(mode="promise_in_bounds")
                return jnp.where(lanes == r, a, acc)

            acc = lax.fori_loop(0, 16, body, jnp.zeros((16,), jnp.float32))
            out_v[pl.ds(b0 + gbase, 16)] = 0.5 * acc
            return carry
        lax.fori_loop(0, _CH // 16, group, 0)

    bufs = ((idx0, rows0, sem0), (idx1, rows1, sem1))
    handles = load_and_fire(0, *bufs[0])
    prev_rows = rows0
    for g in range(1, _NCHUNK):
        idxb, rows, sem = bufs[g % 2]
        new_handles = load_and_fire(g, idxb, rows, sem)
        for h in handles:
            h.wait()
        compute(g - 1, prev_rows)
        handles, prev_rows = new_handles, rows
    for h in handles:
        h.wait()
    compute(_NCHUNK - 1, prev_rows)

    pltpu.sync_copy(out_v, out_hbm.at[pl.ds(wid * _BPW, _BPW)])


def kernel(x, table):
    pat = (np.arange(_IPC, dtype=np.int64) % _F) * _FIELD_DIM
    pat = jnp.asarray(pat.astype(np.int32))
    x2 = x.reshape(_B * _F)

    mesh = plsc.VectorSubcoreMesh(core_axis_name="c", subcore_axis_name="s")
    fm = functools.partial(
        pl.kernel,
        mesh=mesh,
        compiler_params=pltpu.CompilerParams(use_tc_tiling_on_sc=False),
        out_type=jax.ShapeDtypeStruct((_B,), jnp.float32),
        scratch_types=[
            pltpu.VMEM((_IPC,), jnp.int32),
            pltpu.VMEM((_NG, _GW), jnp.int32),
            pltpu.VMEM((_NG, _GW), jnp.int32),
            pltpu.VMEM((_IPC, _D), jnp.float32),
            pltpu.VMEM((_IPC, _D), jnp.float32),
            pltpu.VMEM((_IPC,), jnp.int32),
            pltpu.VMEM((_BPW,), jnp.float32),
            pltpu.SemaphoreType.DMA,
            pltpu.SemaphoreType.DMA,
        ],
    )(_fm_body)
    return fm(x2, pat, table)


# speedup vs baseline: 3.3358x; 2.6685x over previous
import functools

import numpy as np
import jax
import jax.numpy as jnp
from jax import lax
from jax.experimental import pallas as pl
from jax.experimental.pallas import tpu as pltpu
from jax.experimental.pallas import tpu_sc as plsc

_FIELD_DIM = 38462
_F = 26
_D = 16
_B = 16384
_R = 1000012
_NC = 2
_NS = 16
_NW = _NC * _NS

_SLAB = 256
_SPW = 122
_MAIN = _NW * _SPW * _SLAB
_TAIL = _R - _MAIN

_BPW = _B // _NW
_CH = 64
_NCHUNK = _BPW // _CH
_IPC = _CH * _F
_GW = 128
_NG = _IPC // _GW


def _relayout_body(tt_hbm, tail_hbm, out_hbm,
                   slab0, slab1, stage0, stage1, tailv,
                   sin0, sin1, sout0, sout1):
    wid = lax.axis_index("s") * _NC + lax.axis_index("c")
    lanes = lax.iota(jnp.int32, 16)
    base = wid * _SPW
    slabs = (slab0, slab1)
    stages = (stage0, stage1)
    sins = (sin0, sin1)
    souts = (sout0, sout1)

    def start_in(s, b):
        return pltpu.async_copy(
            tt_hbm.at[:, pl.ds((base + s) * _SLAB, _SLAB)], slabs[b], sins[b])

    def start_out(s, b):
        return pltpu.async_copy(
            stages[b], out_hbm.at[pl.ds((base + s) * _SLAB * _D, _SLAB * _D)],
            souts[b])

    start_in(0, 0)
    start_in(1, 1)

    def pair(i, carry):
        for b in range(2):
            s = i * 2 + b
            pltpu.make_async_copy(
                tt_hbm.at[:, pl.ds(0, _SLAB)], slabs[b], sins[b]).wait()

            @pl.when(s >= 2)
            def _():
                pltpu.make_async_copy(
                    stages[b], out_hbm.at[pl.ds(0, _SLAB * _D)],
                    souts[b]).wait()

            for cg in range(_SLAB // 16):
                for d in range(_D):
                    v = slabs[b][d, pl.ds(cg * 16, 16)]
                    plsc.store_scatter(
                        stages[b], [cg * 256 + lanes * _D + d], v)

            start_out(s, b)

            @pl.when(s + 2 < _SPW)
            def _():
                start_in(s + 2, b)
        return carry

    lax.fori_loop(0, _SPW // 2, pair, 0)
    for b in range(2):
        pltpu.make_async_copy(
            stages[b], out_hbm.at[pl.ds(0, _SLAB * _D)], souts[b]).wait()

    @pl.when(wid == 0)
    def _():
        pltpu.sync_copy(tail_hbm, tailv)
        pltpu.sync_copy(tailv, out_hbm.at[pl.ds(_MAIN * _D, _TAIL * _D)])


def _fm_body(x_hbm, pat_hbm, table_hbm, out_hbm,
             xbuf, idx0, idx1, rows0, rows1, patv, out_v, sem0, sem1):
    wid = lax.axis_index("s") * _NC + lax.axis_index("c")
    x0 = wid * (_IPC * _NCHUNK)

    pltpu.sync_copy(pat_hbm, patv)

    def load_and_fire(g, idxb, rows, sem):
        pltpu.sync_copy(x_hbm.at[pl.ds(x0 + g * _IPC, _IPC)], xbuf)

        def add_off(j, carry):
            for v in range(_GW // 16):
                sl = pl.ds(j * _GW + v * 16, 16)
                idxb[j, pl.ds(v * 16, 16)] = xbuf[sl] + patv[sl]
            return carry
        lax.fori_loop(0, _NG, add_off, 0)

        handles = []
        for j in range(_NG):
            handles.append(
                pltpu.async_copy(table_hbm.at[idxb.at[j]],
                                 rows.at[pl.ds(j * _GW, _GW), :], sem))
        return handles

    def compute(g, rows):
        b0 = g * _CH
        lanes = lax.iota(jnp.int32, 16)

        def group(t, carry):
            gbase = t * 16

            def body(r, acc):
                base = (gbase + r) * _F
                e = rows[base, :]
                s = e
                q = e * e
                for f in range(1, _F):
                    e = rows[base + f, :]
                    s = s + e
                    q = q + e * e
                a = s * s - q
                for sh in (8, 4, 2, 1):
                    a = a + a.at[lanes ^ sh].get(mode="promise_in_bounds")
                return jnp.where(lanes == r, a, acc)

            acc = lax.fori_loop(0, 16, body, jnp.zeros((16,), jnp.float32))
            out_v[pl.ds(b0 + gbase, 16)] = 0.5 * acc
            return carry
        lax.fori_loop(0, _CH // 16, group, 0)

    bufs = ((idx0, rows0, sem0), (idx1, rows1, sem1))
    handles = load_and_fire(0, *bufs[0])
    prev_rows = rows0
    for g in range(1, _NCHUNK):
        idxb, rows, sem = bufs[g % 2]
        new_handles = load_and_fire(g, idxb, rows, sem)
        for h in handles:
            h.wait()
        compute(g - 1, prev_rows)
        handles, prev_rows = new_handles, rows
    for h in handles:
        h.wait()
    compute(_NCHUNK - 1, prev_rows)

    pltpu.sync_copy(out_v, out_hbm.at[pl.ds(wid * _BPW, _BPW)])


def kernel(x, table):
    mesh = plsc.VectorSubcoreMesh(core_axis_name="c", subcore_axis_name="s")

    relayout = functools.partial(
        pl.kernel,
        mesh=mesh,
        compiler_params=pltpu.CompilerParams(use_tc_tiling_on_sc=True,
                                             needs_layout_passes=False),
        out_type=jax.ShapeDtypeStruct((_R * _D,), jnp.float32),
        scratch_types=[
            pltpu.VMEM((_D, _SLAB), jnp.float32),
            pltpu.VMEM((_D, _SLAB), jnp.float32),
            pltpu.VMEM((_SLAB * _D,), jnp.float32),
            pltpu.VMEM((_SLAB * _D,), jnp.float32),
            pltpu.VMEM((_TAIL * _D,), jnp.float32),
            pltpu.SemaphoreType.DMA,
            pltpu.SemaphoreType.DMA,
            pltpu.SemaphoreType.DMA,
            pltpu.SemaphoreType.DMA,
        ],
    )(_relayout_body)

    fm = functools.partial(
        pl.kernel,
        mesh=mesh,
        compiler_params=pltpu.CompilerParams(use_tc_tiling_on_sc=False),
        out_type=jax.ShapeDtypeStruct((_B,), jnp.float32),
        scratch_types=[
            pltpu.VMEM((_IPC,), jnp.int32),
            pltpu.VMEM((_NG, _GW), jnp.int32),
            pltpu.VMEM((_NG, _GW), jnp.int32),
            pltpu.VMEM((_IPC, _D), jnp.float32),
            pltpu.VMEM((_IPC, _D), jnp.float32),
            pltpu.VMEM((_IPC,), jnp.int32),
            pltpu.VMEM((_BPW,), jnp.float32),
            pltpu.SemaphoreType.DMA,
            pltpu.SemaphoreType.DMA,
        ],
    )(_fm_body)

    tail = table[_MAIN:, :].reshape(_TAIL * _D)
    t1d = relayout(table.T, tail)
    t2d = t1d.reshape(_R, _D)

    pat = (np.arange(_IPC, dtype=np.int64) % _F) * _FIELD_DIM
    pat = jnp.asarray(pat.astype(np.int32))
    x2 = x.reshape(_B * _F)
    return fm(x2, pat, t2d)
